# baseline (speedup 1.0000x reference)
import functools

import jax
import jax.numpy as jnp
from jax import lax
from jax.experimental import pallas as pl
from jax.experimental.pallas import tpu as pltpu
from jax.experimental.pallas import tpu_sc as plsc

NC = 2
NS = 16
NW = NC * NS
KR = 2


def _mesh():
    return plsc.VectorSubcoreMesh(core_axis_name="c", subcore_axis_name="s",
                                  num_cores=NC, num_subcores=NS)


def _lane_sum(v, lane):
    for sh in (8, 4, 2, 1):
        v = v + lax.gather(
            v, (lane ^ sh)[:, None],
            lax.GatherDimensionNumbers(
                offset_dims=(), collapsed_slice_dims=(0,),
                start_index_map=(0,)),
            slice_sizes=(1,),
            mode=lax.GatherScatterMode.PROMISE_IN_BOUNDS)
    return v


def _perm(v, idx):
    return lax.gather(
        v, idx[:, None],
        lax.GatherDimensionNumbers(
            offset_dims=(), collapsed_slice_dims=(0,), start_index_map=(0,)),
        slice_sizes=(1,),
        mode=lax.GatherScatterMode.PROMISE_IN_BOUNDS)


def _relayout(oh_tabT, oh_tail, mh_tabT, mh_tail):
    D, N1 = oh_tabT.shape
    N2 = mh_tabT.shape[1]
    nf1, nf2 = N1 // 128, N2 // 128
    cnt1, cnt2 = -(-nf1 // NW), -(-nf2 // NW)

    @functools.partial(
        pl.kernel,
        out_type=(
            jax.ShapeDtypeStruct(((nf1 + 1) * 16, 128), jnp.float32),
            jax.ShapeDtypeStruct(((nf2 + 3) * 16, 128), jnp.float32),
        ),
        mesh=_mesh(),
        compiler_params=pltpu.CompilerParams(use_tc_tiling_on_sc=True),
        scratch_types=[
            pltpu.VMEM((4, D, 128), jnp.float32),
            pltpu.VMEM((4, 16, 128), jnp.float32),
            pltpu.SemaphoreType.DMA((4,)),
            pltpu.SemaphoreType.DMA((4,)),
        ],
    )
    def k1(t1_h, tl1_h, t2_h, tl2_h, o1_h, o2_h, inb, outb, rsem, wsem):
        wid = lax.axis_index("s") * NC + lax.axis_index("c")
        lane = lax.iota(jnp.int32, 16)
        masks = [(lane & s) == 0 for s in (1, 2, 4, 8)]
        perms = [lane ^ s for s in (1, 2, 4, 8)]

        def transpose(b):
            for blk in range(8):
                vs = [inb[b, d, pl.ds(blk * 16, 16)] for d in range(D)]
                for st in range(4):
                    s = 1 << st
                    for i in range(16):
                        if i & s:
                            continue
                        a, bb = vs[i], vs[i | s]
                        vs[i] = jnp.where(masks[st], a, _perm(bb, perms[st]))
                        vs[i | s] = jnp.where(masks[st], _perm(a, perms[st]),
                                              bb)
                for c in range(16):
                    outb[b, 2 * blk + (c >> 3), pl.ds((c & 7) * 16, 16)] = \
                        vs[c]

        def phase(tab_h, out_h, nfull, cnt):
            def slab_idx(k):
                return jnp.minimum(wid + k * NW, nfull - 1)

            def fire_read(k, b):
                pltpu.async_copy(tab_h.at[:, pl.ds(slab_idx(k) * 128, 128)],
                                 inb.at[b], rsem.at[b])

            for b in range(4):
                fire_read(b, b)

            def body(it, _):
                b = it & 3
                pltpu.make_async_copy(tab_h.at[:, pl.ds(0, 128)],
                                      inb.at[b], rsem.at[b]).wait()

                @pl.when(it >= 4)
                def _():
                    pltpu.make_async_copy(outb.at[b],
                                          out_h.at[pl.ds(0, 16)],
                                          wsem.at[b]).wait()

                transpose(b)
                pltpu.async_copy(outb.at[b],
                                 out_h.at[pl.ds(slab_idx(it) * 16, 16)],
                                 wsem.at[b])
                fire_read(it + 4, b)
                return 0

            lax.fori_loop(0, cnt, body, 0)
            for b in range(4):
                pltpu.make_async_copy(tab_h.at[:, pl.ds(0, 128)],
                                      inb.at[b], rsem.at[b]).wait()
                pltpu.make_async_copy(outb.at[b], out_h.at[pl.ds(0, 16)],
                                      wsem.at[b]).wait()

        phase(t1_h, o1_h, nf1, cnt1)
        phase(t2_h, o2_h, nf2, cnt2)

        @pl.when(wid == 0)
        def _():
            pltpu.sync_copy(tl1_h, inb.at[0])
            transpose(0)
            pltpu.sync_copy(outb.at[0], o1_h.at[pl.ds(nf1 * 16, 16)])
            pltpu.sync_copy(tl2_h, inb.at[1])
            transpose(1)
            pltpu.sync_copy(outb.at[1], o2_h.at[pl.ds(nf2 * 16, 16)])

    return k1(oh_tabT, oh_tail, mh_tabT, mh_tail)


@functools.partial(jax.jit, static_argnames=("V",))
def _sc_lr(oh_idsT, mh_idsT, dense_pad, oh_tabT, mh_tabT, oh_tail, mh_tail,
           w_all, V):
    F, B = oh_idsT.shape
    L = mh_idsT.shape[0]
    D = dense_pad.shape[1]
    NF = F + L
    SPW = B // NW

    oh_g, mh_g = _relayout(oh_tabT, oh_tail, mh_tabT, mh_tail)

    CH = 32
    NCH = SPW // CH
    R2 = mh_g.shape[0]
    SCH = R2 // NS

    @functools.partial(
        pl.kernel,
        out_type=jax.ShapeDtypeStruct((B,), jnp.float32),
        mesh=_mesh(),
        compiler_params=pltpu.CompilerParams(use_tc_tiling_on_sc=True,
                                             internal_scratch_in_bytes=24576),
        scratch_types=[
            pltpu.VMEM((NF, SPW), jnp.int32),
            pltpu.VMEM((KR * CH,), jnp.int32),
            pltpu.VMEM((KR, CH, 128), jnp.float32),
            pltpu.VMEM((SPW // 8, 128), jnp.float32),
            pltpu.VMEM((SPW // 8, 128), jnp.float32),
            pltpu.VMEM((4, 128), jnp.float32),
            pltpu.VMEM((SPW,), jnp.float32),
            pltpu.VMEM_SHARED((R2, 128), jnp.float32),
            pltpu.SemaphoreType.DMA((KR,)),
            pltpu.SemaphoreType.DMA,
        ],
    )
    def k2(oh_idsT_h, mh_idsT_h, dense_h, oh_g_h, mh_g_h, w_h, out_h,
           p_all, g_cur, ring, acc, dense_v, w_v, out_v, mh_sp,
           sems, ssem):
        sid = lax.axis_index("s")
        wid = sid * NC + lax.axis_index("c")
        base = wid * SPW

        ssl = pl.ds(sid * SCH, SCH)
        stage = pltpu.async_copy(mh_g_h.at[ssl], mh_sp.at[ssl], ssem)

        pltpu.sync_copy(oh_idsT_h.at[:, pl.ds(base, SPW)],
                        p_all.at[pl.ds(0, F)])
        pltpu.sync_copy(mh_idsT_h.at[:, pl.ds(base, SPW)],
                        p_all.at[pl.ds(F, L)])
        pltpu.sync_copy(dense_h.at[pl.ds(wid * (SPW // 8), SPW // 8)],
                        dense_v)
        pltpu.sync_copy(w_h, w_v)

        def flat_j(j, _):
            def flat_i(i, _):
                sl = pl.ds(i * 16, 16)
                p_all[j, sl] = p_all[j, sl] + j * V
                return 0

            return lax.fori_loop(0, SPW // 16, flat_i, 0)

        lax.fori_loop(0, F, flat_j, 0)

        wd = w_v[(F + 1) >> 3, pl.ds(((F + 1) & 7) * 16, 16)]
        brow = w_v[(F + 2) >> 3, pl.ds(((F + 2) & 7) * 16, 16)]
        wm = w_v[F >> 3, pl.ds((F & 7) * 16, 16)] * (1.0 / L)

        def init_s(s, _):
            sl = pl.ds((s & 7) * 16, 16)
            acc[s >> 3, sl] = dense_v[s >> 3, sl] * wd + brow
            return 0

        lax.fori_loop(0, SPW, init_s, 0)

        stage.wait()
        plsc.subcore_barrier()

        def fire(c, j, b):
            for i in range(CH // 16):
                g_cur[pl.ds(b * CH + i * 16, 16)] = \
                    p_all[j, pl.ds(c * CH + i * 16, 16)] >> 3

            @pl.when(j < F)
            def _():
                pltpu.async_copy(oh_g_h.at[g_cur.at[pl.ds(b * CH, CH)]],
                                 ring.at[b], sems.at[b])

            @pl.when(jnp.logical_and(j >= F, j < NF))
            def _():
                pltpu.async_copy(mh_sp.at[g_cur.at[pl.ds(b * CH, CH)]],
                                 ring.at[b], sems.at[b])

        def wait_ring(b):
            pltpu.make_async_copy(oh_g_h.at[g_cur.at[pl.ds(0, CH)]],
                                  ring.at[b], sems.at[b]).wait()

        def compute(c, j, b):
            jc = jnp.minimum(j, F)
            wrow = jnp.where(j < F,
                             w_v[jc >> 3, pl.ds((jc & 7) * 16, 16)], wm)

            def acc_g(g, _):
                sl = pl.ds(c * CH + g * 16, 16)
                ov = (p_all[j, sl] & 7) << 4
                row0 = (c * CH + g * 16) >> 3
                for k in range(16):
                    asl = pl.ds((k & 7) * 16, 16)
                    acc[row0 + (k >> 3), asl] = (
                        acc[row0 + (k >> 3), asl]
                        + ring[b, g * 16 + k, pl.ds(ov[k], 16)] * wrow)
                return 0

            lax.fori_loop(0, CH // 16, acc_g, 0)

        for c in range(NCH):
            for b in range(KR):
                fire(c, b, b)

            def ring_j(jj, _, c=c):
                for b in range(KR):
                    j = jj * KR + b
                    wait_ring(b)
                    compute(c, j, b)

                    @pl.when(j + KR < NF)
                    def _(j=j, b=b, c=c):
                        fire(c, j + KR, b)

                return 0

            lax.fori_loop(0, NF // KR, ring_j, 0)

        lane = lax.iota(jnp.int32, 16)

        def red_g(g, _):
            def red_s(j, tv):
                v = acc[g * 2 + (j >> 3), pl.ds((j & 7) * 16, 16)]
                return jnp.where(lane == j, _lane_sum(v, lane), tv)

            tv = lax.fori_loop(0, 16, red_s, jnp.zeros((16,), jnp.float32))
            out_v[pl.ds(g * 16, 16)] = 1.0 / (1.0 + jnp.exp(-tv))
            return 0

        lax.fori_loop(0, SPW // 16, red_g, 0)
        pltpu.sync_copy(out_v, out_h.at[pl.ds(base, SPW)])

    return k2(oh_idsT, mh_idsT, dense_pad, oh_g, mh_g, w_all)


def kernel(one_hot_ids, multi_hot_ids, dense_feats, one_hot_tables,
           multi_hot_table, W, b):
    B, F = one_hot_ids.shape
    V, D = multi_hot_table.shape
    FV = F * V
    DD = dense_feats.shape[1]
    oh_idsT = one_hot_ids.T
    mh_idsT = multi_hot_ids.T
    oh_tabT = one_hot_tables.T
    mh_tabT = multi_hot_table.T
    oh_tail = jnp.pad(oh_tabT[:, (FV // 128) * 128:],
                      ((0, 0), (0, 128 - FV % 128)))
    mh_tail = jnp.pad(mh_tabT[:, (V // 128) * 128:],
                      ((0, 0), (0, 128 - V % 128)))
    dense_pad = jnp.pad(dense_feats,
                        ((0, 0), (0, D - DD))).reshape(B // 8, 8 * D)
    w = W[:, 0]
    w_oh = w[: F * D].reshape(F, D)
    w_mh = w[F * D: F * D + D].reshape(1, D)
    w_dn = jnp.pad(w[F * D + D:], (0, D - DD)).reshape(1, D)
    brow = jnp.pad(b.astype(jnp.float32), (0, D - 1)).reshape(1, D)
    w_all = jnp.concatenate(
        [w_oh, w_mh, w_dn, brow, jnp.zeros((32 - F - 3, D), jnp.float32)],
        0).reshape(4, 8 * D)
    out = _sc_lr(oh_idsT, mh_idsT, dense_pad, oh_tabT, mh_tabT,
                 oh_tail, mh_tail, w_all, V=V)
    return out.reshape(B, 1)


# speedup vs baseline: 1.1158x; 1.1158x over previous
import functools

import jax
import jax.numpy as jnp
from jax import lax
from jax.experimental import pallas as pl
from jax.experimental.pallas import tpu as pltpu
from jax.experimental.pallas import tpu_sc as plsc

NC = 2
NS = 16
NW = NC * NS
KR = 4


def _mesh():
    return plsc.VectorSubcoreMesh(core_axis_name="c", subcore_axis_name="s",
                                  num_cores=NC, num_subcores=NS)


def _lane_sum(v, lane):
    for sh in (8, 4, 2, 1):
        v = v + lax.gather(
            v, (lane ^ sh)[:, None],
            lax.GatherDimensionNumbers(
                offset_dims=(), collapsed_slice_dims=(0,),
                start_index_map=(0,)),
            slice_sizes=(1,),
            mode=lax.GatherScatterMode.PROMISE_IN_BOUNDS)
    return v


def _perm(v, idx):
    return lax.gather(
        v, idx[:, None],
        lax.GatherDimensionNumbers(
            offset_dims=(), collapsed_slice_dims=(0,), start_index_map=(0,)),
        slice_sizes=(1,),
        mode=lax.GatherScatterMode.PROMISE_IN_BOUNDS)


def _relayout(oh_tabT, oh_tail, mh_tabT, mh_tail):
    D, N1 = oh_tabT.shape
    N2 = mh_tabT.shape[1]
    nf1, nf2 = N1 // 128, N2 // 128
    cnt1, cnt2 = -(-nf1 // NW), -(-nf2 // NW)

    @functools.partial(
        pl.kernel,
        out_type=(
            jax.ShapeDtypeStruct(((nf1 + 1) * 16, 128), jnp.float32),
            jax.ShapeDtypeStruct(((nf2 + 1) * 16, 128), jnp.float32),
        ),
        mesh=_mesh(),
        compiler_params=pltpu.CompilerParams(use_tc_tiling_on_sc=True),
        scratch_types=[
            pltpu.VMEM((4, D, 128), jnp.float32),
            pltpu.VMEM((4, 16, 128), jnp.float32),
            pltpu.SemaphoreType.DMA((4,)),
            pltpu.SemaphoreType.DMA((4,)),
        ],
    )
    def k1(t1_h, tl1_h, t2_h, tl2_h, o1_h, o2_h, inb, outb, rsem, wsem):
        wid = lax.axis_index("s") * NC + lax.axis_index("c")
        lane = lax.iota(jnp.int32, 16)
        masks = [(lane & s) == 0 for s in (1, 2, 4, 8)]
        perms = [lane ^ s for s in (1, 2, 4, 8)]

        def transpose(b):
            for blk in range(8):
                vs = [inb[b, d, pl.ds(blk * 16, 16)] for d in range(D)]
                for st in range(4):
                    s = 1 << st
                    for i in range(16):
                        if i & s:
                            continue
                        a, bb = vs[i], vs[i | s]
                        vs[i] = jnp.where(masks[st], a, _perm(bb, perms[st]))
                        vs[i | s] = jnp.where(masks[st], _perm(a, perms[st]),
                                              bb)
                for c in range(16):
                    outb[b, 2 * blk + (c >> 3), pl.ds((c & 7) * 16, 16)] = \
                        vs[c]

        def phase(tab_h, out_h, nfull, cnt):
            def slab_idx(k):
                return jnp.minimum(wid + k * NW, nfull - 1)

            def fire_read(k, b):
                pltpu.async_copy(tab_h.at[:, pl.ds(slab_idx(k) * 128, 128)],
                                 inb.at[b], rsem.at[b])

            for b in range(4):
                fire_read(b, b)

            def body(it, _):
                b = it & 3
                pltpu.make_async_copy(tab_h.at[:, pl.ds(0, 128)],
                                      inb.at[b], rsem.at[b]).wait()

                @pl.when(it >= 4)
                def _():
                    pltpu.make_async_copy(outb.at[b],
                                          out_h.at[pl.ds(0, 16)],
                                          wsem.at[b]).wait()

                transpose(b)
                pltpu.async_copy(outb.at[b],
                                 out_h.at[pl.ds(slab_idx(it) * 16, 16)],
                                 wsem.at[b])
                fire_read(it + 4, b)
                return 0

            lax.fori_loop(0, cnt, body, 0)
            for b in range(4):
                pltpu.make_async_copy(tab_h.at[:, pl.ds(0, 128)],
                                      inb.at[b], rsem.at[b]).wait()
                pltpu.make_async_copy(outb.at[b], out_h.at[pl.ds(0, 16)],
                                      wsem.at[b]).wait()

        phase(t1_h, o1_h, nf1, cnt1)
        phase(t2_h, o2_h, nf2, cnt2)

        @pl.when(wid == 0)
        def _():
            pltpu.sync_copy(tl1_h, inb.at[0])
            transpose(0)
            pltpu.sync_copy(outb.at[0], o1_h.at[pl.ds(nf1 * 16, 16)])
            pltpu.sync_copy(tl2_h, inb.at[1])
            transpose(1)
            pltpu.sync_copy(outb.at[1], o2_h.at[pl.ds(nf2 * 16, 16)])

    return k1(oh_tabT, oh_tail, mh_tabT, mh_tail)


@functools.partial(jax.jit, static_argnames=("V",))
def _sc_lr(oh_idsT, mh_idsT, dense_pad, oh_tabT, mh_tabT, oh_tail, mh_tail,
           w_all, V):
    F, B = oh_idsT.shape
    L = mh_idsT.shape[0]
    D = dense_pad.shape[1]
    NF = F + L
    SPW = B // NW

    oh_g, mh_g = _relayout(oh_tabT, oh_tail, mh_tabT, mh_tail)

    @functools.partial(
        pl.kernel,
        out_type=jax.ShapeDtypeStruct((B,), jnp.float32),
        mesh=_mesh(),
        compiler_params=pltpu.CompilerParams(use_tc_tiling_on_sc=True),
        scratch_types=[
            pltpu.VMEM((NF, SPW), jnp.int32),
            pltpu.VMEM((NF, SPW), jnp.int32),
            pltpu.VMEM((KR, SPW, 128), jnp.float32),
            pltpu.VMEM((SPW, D), jnp.float32),
            pltpu.VMEM((SPW, D), jnp.float32),
            pltpu.VMEM((NF + 8, D), jnp.float32),
            pltpu.VMEM((SPW,), jnp.float32),
            pltpu.SemaphoreType.DMA,
            pltpu.SemaphoreType.DMA,
            pltpu.SemaphoreType.DMA,
            pltpu.SemaphoreType.DMA,
        ],
    )
    def k2(oh_idsT_h, mh_idsT_h, dense_h, oh_g_h, mh_g_h, w_h, out_h,
           g_all, o_all, ring, acc, dense_v, w_v, out_v,
           sem0, sem1, sem2, sem3):
        wid = lax.axis_index("s") * NC + lax.axis_index("c")
        base = wid * SPW
        sems = (sem0, sem1, sem2, sem3)

        pltpu.sync_copy(oh_idsT_h.at[:, pl.ds(base, SPW)],
                        g_all.at[pl.ds(0, F)])
        pltpu.sync_copy(mh_idsT_h.at[:, pl.ds(base, SPW)],
                        g_all.at[pl.ds(F, L)])
        pltpu.sync_copy(dense_h.at[pl.ds(base, SPW)], dense_v)
        pltpu.sync_copy(w_h, w_v.at[pl.ds(0, 32)])

        def split_j(j, _):
            off = jnp.where(j < F, j * V, 0)

            def split_i(i, _):
                sl = pl.ds(i * 16, 16)
                t = g_all[j, sl] + off
                o_all[j, sl] = (t & 7) << 4
                g_all[j, sl] = t >> 3
                return 0

            return lax.fori_loop(0, SPW // 16, split_i, 0)

        lax.fori_loop(0, NF, split_j, 0)

        wd = w_v[F + 1]
        brow = w_v[F + 2]
        wm = w_v[F] * (1.0 / L)

        def wfill(j, _):
            w_v[j] = wm
            return 0

        lax.fori_loop(F, NF, wfill, 0)

        def init_s(s, _):
            acc[s] = dense_v[s] * wd + brow
            return 0

        lax.fori_loop(0, SPW, init_s, 0)

        def fire(j, b):
            @pl.when(j < F)
            def _():
                pltpu.async_copy(oh_g_h.at[g_all.at[j]], ring.at[b], sems[b])

            @pl.when(jnp.logical_and(j >= F, j < NF))
            def _():
                pltpu.async_copy(mh_g_h.at[g_all.at[j]], ring.at[b], sems[b])

        def wait_ring(b):
            pltpu.make_async_copy(oh_g_h.at[g_all.at[0]], ring.at[b],
                                  sems[b]).wait()

        def compute(j, b):
            wrow = w_v[j]

            def acc_g(g, _):
                ov = o_all[j, pl.ds(g * 16, 16)]
                for k in range(16):
                    s = g * 16 + k
                    acc[s] = acc[s] + ring[b, s, pl.ds(ov[k], 16)] * wrow
                return 0

            lax.fori_loop(0, SPW // 16, acc_g, 0)

        for b in range(KR):
            fire(b, b)

        def ring_j(jj, _):
            for b in range(KR):
                j = jj * KR + b
                wait_ring(b)
                compute(j, b)
                fire(j + KR, b)
            return 0

        lax.fori_loop(0, NF // KR, ring_j, 0)

        lane = lax.iota(jnp.int32, 16)

        def red_g(g, _):
            def red_s(j, tv):
                return jnp.where(lane == j, _lane_sum(acc[g * 16 + j], lane),
                                 tv)

            tv = lax.fori_loop(0, 16, red_s, jnp.zeros((16,), jnp.float32))
            out_v[pl.ds(g * 16, 16)] = 1.0 / (1.0 + jnp.exp(-tv))
            return 0

        lax.fori_loop(0, SPW // 16, red_g, 0)
        pltpu.sync_copy(out_v, out_h.at[pl.ds(base, SPW)])

    return k2(oh_idsT, mh_idsT, dense_pad, oh_g, mh_g, w_all)


def kernel(one_hot_ids, multi_hot_ids, dense_feats, one_hot_tables,
           multi_hot_table, W, b):
    B, F = one_hot_ids.shape
    V, D = multi_hot_table.shape
    FV = F * V
    DD = dense_feats.shape[1]
    oh_idsT = one_hot_ids.T
    mh_idsT = multi_hot_ids.T
    oh_tabT = one_hot_tables.T
    mh_tabT = multi_hot_table.T
    oh_tail = jnp.pad(oh_tabT[:, (FV // 128) * 128:],
                      ((0, 0), (0, 128 - FV % 128)))
    mh_tail = jnp.pad(mh_tabT[:, (V // 128) * 128:],
                      ((0, 0), (0, 128 - V % 128)))
    dense_pad = jnp.pad(dense_feats, ((0, 0), (0, D - DD)))
    w = W[:, 0]
    w_oh = w[: F * D].reshape(F, D)
    w_mh = w[F * D: F * D + D].reshape(1, D)
    w_dn = jnp.pad(w[F * D + D:], (0, D - DD)).reshape(1, D)
    brow = jnp.pad(b.astype(jnp.float32), (0, D - 1)).reshape(1, D)
    w_all = jnp.concatenate(
        [w_oh, w_mh, w_dn, brow, jnp.zeros((32 - F - 3, D), jnp.float32)], 0)
    out = _sc_lr(oh_idsT, mh_idsT, dense_pad, oh_tabT, mh_tabT,
                 oh_tail, mh_tail, w_all, V=V)
    return out.reshape(B, 1)


# speedup vs baseline: 1.1702x; 1.0488x over previous
import functools

import jax
import jax.numpy as jnp
from jax import lax
from jax.experimental import pallas as pl
from jax.experimental.pallas import tpu as pltpu
from jax.experimental.pallas import tpu_sc as plsc

NC = 2
NS = 16
NW = NC * NS
KR = 4


def _mesh():
    return plsc.VectorSubcoreMesh(core_axis_name="c", subcore_axis_name="s",
                                  num_cores=NC, num_subcores=NS)


def _lane_sum(v, lane):
    for sh in (8, 4, 2, 1):
        v = v + lax.gather(
            v, (lane ^ sh)[:, None],
            lax.GatherDimensionNumbers(
                offset_dims=(), collapsed_slice_dims=(0,),
                start_index_map=(0,)),
            slice_sizes=(1,),
            mode=lax.GatherScatterMode.PROMISE_IN_BOUNDS)
    return v


def _perm(v, idx):
    return lax.gather(
        v, idx[:, None],
        lax.GatherDimensionNumbers(
            offset_dims=(), collapsed_slice_dims=(0,), start_index_map=(0,)),
        slice_sizes=(1,),
        mode=lax.GatherScatterMode.PROMISE_IN_BOUNDS)


def _relayout(oh_tabT, oh_tail, mh_tabT, mh_tail):
    D, N1 = oh_tabT.shape
    N2 = mh_tabT.shape[1]
    nf1, nf2 = N1 // 128, N2 // 128
    cnt1, cnt2 = -(-nf1 // NW), -(-nf2 // NW)

    @functools.partial(
        pl.kernel,
        out_type=(
            jax.ShapeDtypeStruct(((nf1 + 1) * 16, 128), jnp.float32),
            jax.ShapeDtypeStruct(((nf2 + 1) * 16, 128), jnp.float32),
        ),
        mesh=_mesh(),
        compiler_params=pltpu.CompilerParams(use_tc_tiling_on_sc=True),
        scratch_types=[
            pltpu.VMEM((8, D, 128), jnp.float32),
            pltpu.VMEM((8, 16, 128), jnp.float32),
            pltpu.SemaphoreType.DMA((8,)),
            pltpu.SemaphoreType.DMA((8,)),
        ],
    )
    def k1(t1_h, tl1_h, t2_h, tl2_h, o1_h, o2_h, inb, outb, rsem, wsem):
        wid = lax.axis_index("s") * NC + lax.axis_index("c")
        lane = lax.iota(jnp.int32, 16)
        masks = [(lane & s) == 0 for s in (1, 2, 4, 8)]
        perms = [lane ^ s for s in (1, 2, 4, 8)]

        def transpose(b):
            for blk in range(8):
                vs = [inb[b, d, pl.ds(blk * 16, 16)] for d in range(D)]
                for st in range(4):
                    s = 1 << st
                    for i in range(16):
                        if i & s:
                            continue
                        a, bb = vs[i], vs[i | s]
                        vs[i] = jnp.where(masks[st], a, _perm(bb, perms[st]))
                        vs[i | s] = jnp.where(masks[st], _perm(a, perms[st]),
                                              bb)
                for c in range(16):
                    outb[b, 2 * blk + (c >> 3), pl.ds((c & 7) * 16, 16)] = \
                        vs[c]

        def phase(tab_h, out_h, nfull, cnt):
            def slab_idx(k):
                return jnp.minimum(wid + k * NW, nfull - 1)

            def fire_read(k, b):
                pltpu.async_copy(tab_h.at[:, pl.ds(slab_idx(k) * 128, 128)],
                                 inb.at[b], rsem.at[b])

            for b in range(8):
                fire_read(b, b)

            def body(it, _):
                b = it & 7
                pltpu.make_async_copy(tab_h.at[:, pl.ds(0, 128)],
                                      inb.at[b], rsem.at[b]).wait()

                @pl.when(it >= 8)
                def _():
                    pltpu.make_async_copy(outb.at[b],
                                          out_h.at[pl.ds(0, 16)],
                                          wsem.at[b]).wait()

                transpose(b)
                pltpu.async_copy(outb.at[b],
                                 out_h.at[pl.ds(slab_idx(it) * 16, 16)],
                                 wsem.at[b])
                fire_read(it + 8, b)
                return 0

            lax.fori_loop(0, cnt, body, 0)
            for b in range(8):
                pltpu.make_async_copy(tab_h.at[:, pl.ds(0, 128)],
                                      inb.at[b], rsem.at[b]).wait()
                pltpu.make_async_copy(outb.at[b], out_h.at[pl.ds(0, 16)],
                                      wsem.at[b]).wait()

        phase(t1_h, o1_h, nf1, cnt1)
        phase(t2_h, o2_h, nf2, cnt2)

        @pl.when(wid == 0)
        def _():
            pltpu.sync_copy(tl1_h, inb.at[0])
            transpose(0)
            pltpu.sync_copy(outb.at[0], o1_h.at[pl.ds(nf1 * 16, 16)])
            pltpu.sync_copy(tl2_h, inb.at[1])
            transpose(1)
            pltpu.sync_copy(outb.at[1], o2_h.at[pl.ds(nf2 * 16, 16)])

    return k1(oh_tabT, oh_tail, mh_tabT, mh_tail)


@functools.partial(jax.jit, static_argnames=("V",))
def _sc_lr(oh_idsT, mh_idsT, dense_pad, oh_tabT, mh_tabT, oh_tail, mh_tail,
           w_all, V):
    F, B = oh_idsT.shape
    L = mh_idsT.shape[0]
    D = dense_pad.shape[1]
    NF = F + L
    SPW = B // NW

    oh_g, mh_g = _relayout(oh_tabT, oh_tail, mh_tabT, mh_tail)

    @functools.partial(
        pl.kernel,
        out_type=jax.ShapeDtypeStruct((B,), jnp.float32),
        mesh=_mesh(),
        compiler_params=pltpu.CompilerParams(use_tc_tiling_on_sc=True),
        scratch_types=[
            pltpu.VMEM((NF, SPW), jnp.int32),
            pltpu.VMEM((NF, SPW), jnp.int32),
            pltpu.VMEM((KR, SPW, 128), jnp.float32),
            pltpu.VMEM((SPW, D), jnp.float32),
            pltpu.VMEM((SPW, D), jnp.float32),
            pltpu.VMEM((NF + 8, D), jnp.float32),
            pltpu.VMEM((SPW,), jnp.float32),
            pltpu.SemaphoreType.DMA,
            pltpu.SemaphoreType.DMA,
            pltpu.SemaphoreType.DMA,
            pltpu.SemaphoreType.DMA,
        ],
    )
    def k2(oh_idsT_h, mh_idsT_h, dense_h, oh_g_h, mh_g_h, w_h, out_h,
           g_all, o_all, ring, acc, dense_v, w_v, out_v,
           sem0, sem1, sem2, sem3):
        wid = lax.axis_index("s") * NC + lax.axis_index("c")
        base = wid * SPW
        sems = (sem0, sem1, sem2, sem3)

        pltpu.sync_copy(oh_idsT_h.at[:, pl.ds(base, SPW)],
                        g_all.at[pl.ds(0, F)])
        pltpu.sync_copy(mh_idsT_h.at[:, pl.ds(base, SPW)],
                        g_all.at[pl.ds(F, L)])
        pltpu.sync_copy(dense_h.at[pl.ds(base, SPW)], dense_v)
        pltpu.sync_copy(w_h, w_v.at[pl.ds(0, 32)])

        def split_j(j, _):
            off = jnp.where(j < F, j * V, 0)

            def split_i(i, _):
                sl = pl.ds(i * 16, 16)
                t = g_all[j, sl] + off
                o_all[j, sl] = (t & 7) << 4
                g_all[j, sl] = t >> 3
                return 0

            return lax.fori_loop(0, SPW // 16, split_i, 0)

        lax.fori_loop(0, NF, split_j, 0)

        wd = w_v[F + 1]
        brow = w_v[F + 2]
        wm = w_v[F] * (1.0 / L)

        def wfill(j, _):
            w_v[j] = wm
            return 0

        lax.fori_loop(F, NF, wfill, 0)

        def init_s(s, _):
            acc[s] = dense_v[s] * wd + brow
            return 0

        lax.fori_loop(0, SPW, init_s, 0)

        def fire(j, b):
            @pl.when(j < F)
            def _():
                pltpu.async_copy(oh_g_h.at[g_all.at[j]], ring.at[b], sems[b])

            @pl.when(jnp.logical_and(j >= F, j < NF))
            def _():
                pltpu.async_copy(mh_g_h.at[g_all.at[j]], ring.at[b], sems[b])

        def wait_ring(b):
            pltpu.make_async_copy(oh_g_h.at[g_all.at[0]], ring.at[b],
                                  sems[b]).wait()

        def compute(j, b):
            wrow = w_v[j]

            def acc_g(g, _):
                ov = o_all[j, pl.ds(g * 16, 16)]
                for k in range(16):
                    s = g * 16 + k
                    acc[s] = acc[s] + ring[b, s, pl.ds(ov[k], 16)] * wrow
                return 0

            lax.fori_loop(0, SPW // 16, acc_g, 0)

        for b in range(KR):
            fire(b, b)

        def ring_j(jj, _):
            for b in range(KR):
                j = jj * KR + b
                wait_ring(b)
                compute(j, b)
                fire(j + KR, b)
            return 0

        lax.fori_loop(0, NF // KR, ring_j, 0)

        lane = lax.iota(jnp.int32, 16)

        def red_g(g, _):
            def red_s(j, tv):
                return jnp.where(lane == j, _lane_sum(acc[g * 16 + j], lane),
                                 tv)

            tv = lax.fori_loop(0, 16, red_s, jnp.zeros((16,), jnp.float32))
            out_v[pl.ds(g * 16, 16)] = 1.0 / (1.0 + jnp.exp(-tv))
            return 0

        lax.fori_loop(0, SPW // 16, red_g, 0)
        pltpu.sync_copy(out_v, out_h.at[pl.ds(base, SPW)])

    return k2(oh_idsT, mh_idsT, dense_pad, oh_g, mh_g, w_all)


def kernel(one_hot_ids, multi_hot_ids, dense_feats, one_hot_tables,
           multi_hot_table, W, b):
    B, F = one_hot_ids.shape
    V, D = multi_hot_table.shape
    FV = F * V
    DD = dense_feats.shape[1]
    oh_idsT = one_hot_ids.T
    mh_idsT = multi_hot_ids.T
    oh_tabT = one_hot_tables.T
    mh_tabT = multi_hot_table.T
    oh_tail = jnp.pad(oh_tabT[:, (FV // 128) * 128:],
                      ((0, 0), (0, 128 - FV % 128)))
    mh_tail = jnp.pad(mh_tabT[:, (V // 128) * 128:],
                      ((0, 0), (0, 128 - V % 128)))
    dense_pad = jnp.pad(dense_feats, ((0, 0), (0, D - DD)))
    w = W[:, 0]
    w_oh = w[: F * D].reshape(F, D)
    w_mh = w[F * D: F * D + D].reshape(1, D)
    w_dn = jnp.pad(w[F * D + D:], (0, D - DD)).reshape(1, D)
    brow = jnp.pad(b.astype(jnp.float32), (0, D - 1)).reshape(1, D)
    w_all = jnp.concatenate(
        [w_oh, w_mh, w_dn, brow, jnp.zeros((32 - F - 3, D), jnp.float32)], 0)
    out = _sc_lr(oh_idsT, mh_idsT, dense_pad, oh_tabT, mh_tabT,
                 oh_tail, mh_tail, w_all, V=V)
    return out.reshape(B, 1)
